# baseline (speedup 1.0000x reference)
import functools

import jax
import jax.numpy as jnp
from jax import lax
from jax.experimental import pallas as pl
from jax.experimental.pallas import tpu as pltpu
from jax.experimental.pallas import tpu_sc as plsc

_LANES = 16
_GATHER = 128


def kernel(bbs_inf, phi_W, psi_W, omega_W):
    B, L, T = bbs_inf.shape
    V, D = phi_W.shape
    R = B * L
    N = R * T

    table = jnp.concatenate([phi_W, psi_W, omega_W], axis=0)
    idx_flat = bbs_inf.reshape(N)

    info = plsc.get_sparse_core_info()
    NW = info.num_cores * info.num_subcores
    per_w = N // NW
    CH = 3 * _GATHER
    n_chunks = per_w // CH

    mesh = plsc.VectorSubcoreMesh(core_axis_name="c", subcore_axis_name="s")

    @functools.partial(
        pl.kernel,
        mesh=mesh,
        out_type=jax.ShapeDtypeStruct((N, D), jnp.float32),
        scratch_types=[
            pltpu.VMEM((2, CH), jnp.int32),
            pltpu.VMEM((2, T, _GATHER), jnp.int32),
            pltpu.VMEM((2, CH, D), jnp.float32),
            pltpu.SemaphoreType.DMA,
            pltpu.SemaphoreType.DMA,
            pltpu.SemaphoreType.DMA,
            pltpu.SemaphoreType.DMA,
        ],
    )
    def k(idx_hbm, w_hbm, out_hbm, idxraw, idxadj, rows, g0, g1, s0, s1):
        wid = lax.axis_index("s") * info.num_cores + lax.axis_index("c")
        base0 = wid * per_w
        iota = lax.iota(jnp.int32, _LANES)
        gsem = (g0, g1)
        ssem = (s0, s1)

        def load_adjust(c, b):
            pltpu.sync_copy(idx_hbm.at[pl.ds(base0 + c * CH, CH)], idxraw.at[b])
            for g in range(CH // _LANES):
                off = ((iota + _LANES * g) % 3) * V
                v = idxraw[b, pl.ds(_LANES * g, _LANES)] + off
                p = _LANES * g
                idxadj[b, p // _GATHER, pl.ds(p % _GATHER, _LANES)] = v

        def fire_gathers(b):
            for j in range(T):
                pltpu.async_copy(
                    w_hbm.at[idxadj.at[b].at[j]],
                    rows.at[b].at[pl.ds(j * _GATHER, _GATHER)],
                    gsem[b],
                )

        def wait_gathers(b):
            for j in range(T):
                pltpu.make_async_copy(
                    w_hbm.at[idxadj.at[b].at[j]],
                    rows.at[b].at[pl.ds(j * _GATHER, _GATHER)],
                    gsem[b],
                ).wait()

        def chunk_op(c, b, prefetch):
            wait_gathers(b)
            sc = pltpu.async_copy(
                rows.at[b], out_hbm.at[pl.ds(base0 + c * CH, CH)], ssem[b]
            )
            if prefetch:
                load_adjust(c + 2, b)
                sc.wait()
                fire_gathers(b)
            else:
                sc.wait()

        for b in range(2):
            load_adjust(b, b)
            fire_gathers(b)

        def body(kk, carry):
            for b in range(2):
                chunk_op(2 * kk + b, b, True)
            return carry

        lax.fori_loop(0, n_chunks // 2 - 1, body, 0)
        for b in range(2):
            chunk_op(n_chunks - 2 + b, b, False)

    out = k(idx_flat, table)
    return out.reshape(B, L, T * D)


# speedup vs baseline: 1.3136x; 1.3136x over previous
import functools

import jax
import jax.numpy as jnp
from jax import lax
from jax.experimental import pallas as pl
from jax.experimental.pallas import tpu as pltpu
from jax.experimental.pallas import tpu_sc as plsc

_LANES = 16
_GATHER = 128


def kernel(bbs_inf, phi_W, psi_W, omega_W):
    B, L, T = bbs_inf.shape
    V, D = phi_W.shape
    R = B * L
    N = R * T

    table = jnp.concatenate([phi_W, psi_W, omega_W], axis=0)
    idx_flat = bbs_inf.reshape(N)

    info = plsc.get_sparse_core_info()
    NW = info.num_cores * info.num_subcores
    per_w = N // NW
    CH = 3 * _GATHER
    n_chunks = per_w // CH

    mesh = plsc.VectorSubcoreMesh(core_axis_name="c", subcore_axis_name="s")

    RPC = CH // T

    @functools.partial(
        pl.kernel,
        mesh=mesh,
        out_type=jax.ShapeDtypeStruct((R, T * D), jnp.float32),
        scratch_types=[
            pltpu.VMEM((2, CH), jnp.int32),
            pltpu.VMEM((2, T, _GATHER), jnp.int32),
            pltpu.VMEM((2, CH, D), jnp.float32),
            pltpu.SemaphoreType.DMA,
            pltpu.SemaphoreType.DMA,
            pltpu.SemaphoreType.DMA,
            pltpu.SemaphoreType.DMA,
        ],
    )
    def k(idx_hbm, w_hbm, out_hbm, idxraw, idxadj, rows, g0, g1, s0, s1):
        wid = lax.axis_index("s") * info.num_cores + lax.axis_index("c")
        base0 = wid * per_w
        rbase0 = wid * (per_w // T)
        iota = lax.iota(jnp.int32, _LANES)
        gsem = (g0, g1)
        ssem = (s0, s1)

        def load_adjust(c, b):
            pltpu.sync_copy(idx_hbm.at[pl.ds(base0 + c * CH, CH)], idxraw.at[b])
            for g in range(CH // _LANES):
                off = ((iota + _LANES * g) % 3) * V
                v = idxraw[b, pl.ds(_LANES * g, _LANES)] + off
                p = _LANES * g
                idxadj[b, p // _GATHER, pl.ds(p % _GATHER, _LANES)] = v

        def fire_gathers(b):
            for j in range(T):
                pltpu.async_copy(
                    w_hbm.at[idxadj.at[b].at[j]],
                    rows.at[b].at[pl.ds(j * _GATHER, _GATHER)],
                    gsem[b],
                )

        def wait_gathers(b):
            for j in range(T):
                pltpu.make_async_copy(
                    w_hbm.at[idxadj.at[b].at[j]],
                    rows.at[b].at[pl.ds(j * _GATHER, _GATHER)],
                    gsem[b],
                ).wait()

        def chunk_op(c, b, prefetch):
            wait_gathers(b)
            sc = pltpu.async_copy(
                rows.at[b].reshape(RPC, T * D),
                out_hbm.at[pl.ds(rbase0 + c * RPC, RPC)],
                ssem[b],
            )
            if prefetch:
                load_adjust(c + 2, b)
                sc.wait()
                fire_gathers(b)
            else:
                sc.wait()

        for b in range(2):
            load_adjust(b, b)
            fire_gathers(b)

        def body(kk, carry):
            for b in range(2):
                chunk_op(2 * kk + b, b, True)
            return carry

        lax.fori_loop(0, n_chunks // 2 - 1, body, 0)
        for b in range(2):
            chunk_op(n_chunks - 2 + b, b, False)

    out = k(idx_flat, table)
    return out.reshape(B, L, T * D)


# speedup vs baseline: 1.3139x; 1.0002x over previous
import functools

import jax
import jax.numpy as jnp
from jax import lax
from jax.experimental import pallas as pl
from jax.experimental.pallas import tpu as pltpu
from jax.experimental.pallas import tpu_sc as plsc

_LANES = 16
_GATHER = 128


def kernel(bbs_inf, phi_W, psi_W, omega_W):
    B, L, T = bbs_inf.shape
    V, D = phi_W.shape
    R = B * L
    N = R * T

    table = jnp.concatenate([phi_W, psi_W, omega_W], axis=0)
    idx_flat = bbs_inf.reshape(N)

    info = plsc.get_sparse_core_info()
    NW = info.num_cores * info.num_subcores
    per_w = N // NW
    CH = 3 * _GATHER
    n_chunks = per_w // CH

    mesh = plsc.VectorSubcoreMesh(core_axis_name="c", subcore_axis_name="s")

    RPC = CH // T

    @functools.partial(
        pl.kernel,
        mesh=mesh,
        out_type=jax.ShapeDtypeStruct((R, T * D), jnp.float32),
        scratch_types=[
            pltpu.VMEM((2, CH), jnp.int32),
            pltpu.VMEM((2, T, _GATHER), jnp.int32),
            pltpu.VMEM((2, CH, D), jnp.float32),
            pltpu.SemaphoreType.DMA,
            pltpu.SemaphoreType.DMA,
            pltpu.SemaphoreType.DMA,
            pltpu.SemaphoreType.DMA,
        ],
        compiler_params=pltpu.CompilerParams(use_tc_tiling_on_sc=True),
    )
    def k(idx_hbm, w_hbm, out_hbm, idxraw, idxadj, rows, g0, g1, s0, s1):
        wid = lax.axis_index("s") * info.num_cores + lax.axis_index("c")
        base0 = wid * per_w
        rbase0 = wid * (per_w // T)
        iota = lax.iota(jnp.int32, _LANES)
        gsem = (g0, g1)
        ssem = (s0, s1)

        def load_adjust(c, b):
            pltpu.sync_copy(idx_hbm.at[pl.ds(base0 + c * CH, CH)], idxraw.at[b])
            for g in range(CH // _LANES):
                off = ((iota + _LANES * g) % 3) * V
                v = idxraw[b, pl.ds(_LANES * g, _LANES)] + off
                p = _LANES * g
                idxadj[b, p // _GATHER, pl.ds(p % _GATHER, _LANES)] = v

        def fire_gathers(b):
            for j in range(T):
                pltpu.async_copy(
                    w_hbm.at[idxadj.at[b].at[j]],
                    rows.at[b].at[pl.ds(j * _GATHER, _GATHER)],
                    gsem[b],
                )

        def wait_gathers(b):
            for j in range(T):
                pltpu.make_async_copy(
                    w_hbm.at[idxadj.at[b].at[j]],
                    rows.at[b].at[pl.ds(j * _GATHER, _GATHER)],
                    gsem[b],
                ).wait()

        def chunk_op(c, b, prefetch):
            wait_gathers(b)
            sc = pltpu.async_copy(
                rows.at[b].reshape(RPC, T * D),
                out_hbm.at[pl.ds(rbase0 + c * RPC, RPC)],
                ssem[b],
            )
            if prefetch:
                load_adjust(c + 2, b)
                sc.wait()
                fire_gathers(b)
            else:
                sc.wait()

        for b in range(2):
            load_adjust(b, b)
            fire_gathers(b)

        def body(kk, carry):
            for b in range(2):
                chunk_op(2 * kk + b, b, True)
            return carry

        lax.fori_loop(0, n_chunks // 2 - 1, body, 0)
        for b in range(2):
            chunk_op(n_chunks - 2 + b, b, False)

    out = k(idx_flat, table)
    return out.reshape(B, L, T * D)
